# Initial kernel scaffold; baseline (speedup 1.0000x reference)
import jax
import jax.numpy as jnp
from jax.experimental import pallas as pl


def kernel(x, gate_w, gate_b, w_gate, w_up, w_down):
    raise NotImplementedError("write your pallas kernel here")


# speedup vs baseline: 1.1016x; 1.1016x over previous
import jax
import jax.numpy as jnp
from jax.experimental import pallas as pl

_E, _D, _FF, _K = 8, 1024, 2048, 2
_FFT = 512


def _routing_kernel(x_ref, gw_ref, gb_ref, logits_ref, gates_ref):
    x = x_ref[...]
    gw = gw_ref[...]
    logits = jax.lax.dot_general(
        x, gw, (((1,), (1,)), ((), ())),
        preferred_element_type=jnp.float32) + gb_ref[...]
    logits_ref[...] = logits
    lane = jax.lax.broadcasted_iota(jnp.int32, logits.shape, 1)
    l0 = jnp.max(logits, axis=1, keepdims=True)
    i0 = jnp.min(jnp.where(logits == l0, lane, _E), axis=1, keepdims=True)
    masked = jnp.where(lane == i0, -jnp.inf, logits)
    l1 = jnp.max(masked, axis=1, keepdims=True)
    i1 = jnp.min(jnp.where(masked == l1, lane, _E), axis=1, keepdims=True)
    g0 = jax.nn.sigmoid(l0 - l1)
    gates_ref[...] = jnp.where(
        lane == i0, g0, jnp.where(lane == i1, 1.0 - g0, 0.0))


def _dense_kernel(gates_ref, x_ref, wg_ref, wu_ref, wd_ref, out_ref):
    e = pl.program_id(0)
    f = pl.program_id(1)
    x = x_ref[...]
    a = jax.lax.dot_general(x, wg_ref[0], (((1,), (1,)), ((), ())),
                            preferred_element_type=jnp.float32)
    b = jax.lax.dot_general(x, wu_ref[0], (((1,), (1,)), ((), ())),
                            preferred_element_type=jnp.float32)
    h = (a * jax.nn.sigmoid(a)) * b
    lane = jax.lax.broadcasted_iota(jnp.int32, gates_ref.shape, 1)
    g = jnp.sum(gates_ref[...] * (lane == e), axis=1, keepdims=True)
    hb = (h * g).astype(jnp.bfloat16)
    contrib = jax.lax.dot_general(hb, wd_ref[0], (((1,), (1,)), ((), ())),
                                  preferred_element_type=jnp.float32)

    @pl.when((e == 0) & (f == 0))
    def _():
        out_ref[...] = jnp.zeros_like(out_ref)

    out_ref[...] += contrib


def kernel(x, gate_w, gate_b, w_gate, w_up, w_down):
    xf = x.reshape(-1, x.shape[-1])
    n = xf.shape[0]
    logits, gates = pl.pallas_call(
        _routing_kernel,
        out_shape=(jax.ShapeDtypeStruct((n, _E), jnp.float32),
                   jax.ShapeDtypeStruct((n, _E), jnp.float32)),
    )(xf, gate_w, gate_b.reshape(1, _E))

    xb = xf.astype(jnp.bfloat16)
    wg = w_gate.astype(jnp.bfloat16)
    wu = w_up.astype(jnp.bfloat16)
    wd = w_down.astype(jnp.bfloat16)
    final = pl.pallas_call(
        _dense_kernel,
        grid=(_E, _FF // _FFT),
        in_specs=[
            pl.BlockSpec((n, _E), lambda e, f: (0, 0)),
            pl.BlockSpec((n, _D), lambda e, f: (0, 0)),
            pl.BlockSpec((1, _FFT, _D), lambda e, f: (e, f, 0)),
            pl.BlockSpec((1, _FFT, _D), lambda e, f: (e, f, 0)),
            pl.BlockSpec((1, _D, _FFT), lambda e, f: (e, 0, f)),
        ],
        out_specs=pl.BlockSpec((n, _D), lambda e, f: (0, 0)),
        out_shape=jax.ShapeDtypeStruct((n, _D), jnp.float32),
    )(gates, xb, wg, wu, wd)
    return final.reshape(x.shape), logits


# speedup vs baseline: 1.2250x; 1.1120x over previous
import jax
import jax.numpy as jnp
from jax.experimental import pallas as pl
from jax.experimental.pallas import tpu as pltpu

_E, _D, _FF, _K = 8, 1024, 2048, 2
_N = 2048
_NK = _N * _K
_TILE = 256
_T = _NK // _TILE
_W = _T + _E - 1
_FFT = 1024
_F = _FF // _FFT


def _cumsum(v, axis):
    n = v.shape[axis]
    sh = 1
    while sh < n:
        z = jnp.zeros_like(jax.lax.slice_in_dim(v, 0, sh, axis=axis))
        v = v + jnp.concatenate(
            [z, jax.lax.slice_in_dim(v, 0, n - sh, axis=axis)], axis=axis)
        sh *= 2
    return v


def _plan_kernel(x_ref, gw_ref, gb_ref,
                 logits_ref, g2_ref, p0_ref, p1_ref, off_ref, cnt_ref):
    x = x_ref[...]
    logits = jax.lax.dot_general(
        x, gw_ref[...], (((1,), (1,)), ((), ())),
        preferred_element_type=jnp.float32) + gb_ref[...]
    logits_ref[...] = logits
    lane = jax.lax.broadcasted_iota(jnp.int32, logits.shape, 1)
    l0 = jnp.max(logits, axis=1, keepdims=True)
    i0 = jnp.min(jnp.where(logits == l0, lane, _E), axis=1, keepdims=True)
    masked = jnp.where(lane == i0, -jnp.inf, logits)
    l1 = jnp.max(masked, axis=1, keepdims=True)
    i1 = jnp.min(jnp.where(masked == l1, lane, _E), axis=1, keepdims=True)
    g0 = jax.nn.sigmoid(l0 - l1)
    g2_ref[...] = jnp.concatenate([g0, 1.0 - g0], axis=1)
    m0 = (lane == i0).astype(jnp.int32)
    m1 = (lane == i1).astype(jnp.int32)
    c0i = _cumsum(m0, 0)
    c1i = _cumsum(m1, 0)
    cnt0 = c0i[_N - 1:_N, :]
    cnt1 = c1i[_N - 1:_N, :]
    counts = cnt0 + cnt1
    off = _cumsum(counts, 1) - counts
    off_ref[...] = off
    cnt_ref[...] = counts
    p0_ref[...] = jnp.sum(m0 * (off + c0i - m0), axis=1, keepdims=True)
    p1_ref[...] = jnp.sum(m1 * (off + cnt0 + c1i - m1), axis=1, keepdims=True)


def _dispatch_kernel(p0_ref, p1_ref, off_ref, cnt_ref,
                     x_ref, xs_ref, wt_ref, we_ref, wv_ref):
    def body(j, carry):
        row = x_ref[j]
        xs_ref[p0_ref[j]] = row
        xs_ref[p1_ref[j]] = row
        return carry
    jax.lax.fori_loop(0, _N, body, 0)

    def wbody(p, idx):
        t = p // _E
        e = p % _E
        s = off_ref[e]
        c = cnt_ref[e]
        active = jnp.logical_and(
            jnp.logical_and(s < (t + 1) * _TILE, s + c > t * _TILE), c > 0)

        @pl.when(active)
        def _():
            wt_ref[idx] = t
            we_ref[idx] = e
            wv_ref[idx] = 1

        return idx + active.astype(jnp.int32)

    nitems = jax.lax.fori_loop(0, _T * _E, wbody, 0)

    def pbody(i, carry):
        @pl.when(i >= nitems)
        def _():
            wt_ref[i] = wt_ref[nitems - 1]
            we_ref[i] = we_ref[nitems - 1]
            wv_ref[i] = 0
        return carry

    jax.lax.fori_loop(0, _W, pbody, 0)


def _ffn_kernel(wt_ref, we_ref, wv_ref, off_ref, cnt_ref,
                xs_ref, wg_ref, wu_ref, wd_ref, out_ref):
    w = pl.program_id(0)
    f = pl.program_id(1)
    t = wt_ref[w]
    e = we_ref[w]
    first = jnp.logical_or(w == 0, t != wt_ref[jnp.maximum(w - 1, 0)])

    @pl.when(jnp.logical_and(first, f == 0))
    def _():
        out_ref[...] = jnp.zeros_like(out_ref)

    @pl.when(wv_ref[w] == 1)
    def _():
        x = xs_ref[...]
        a = jax.lax.dot_general(x, wg_ref[0], (((1,), (1,)), ((), ())),
                                preferred_element_type=jnp.float32)
        b = jax.lax.dot_general(x, wu_ref[0], (((1,), (1,)), ((), ())),
                                preferred_element_type=jnp.float32)
        h = (a * jax.nn.sigmoid(a)) * b
        row = t * _TILE + jax.lax.broadcasted_iota(jnp.int32, (_TILE, 1), 0)
        s = off_ref[e]
        mask = jnp.logical_and(row >= s, row < s + cnt_ref[e])
        hb = jnp.where(mask, h, 0.0).astype(jnp.bfloat16)
        out_ref[...] += jax.lax.dot_general(
            hb, wd_ref[0], (((1,), (1,)), ((), ())),
            preferred_element_type=jnp.float32)


def _combine_kernel(p0_ref, p1_ref, g2_ref, os_ref, out_ref):
    def body(n, carry):
        out_ref[n] = (g2_ref[2 * n] * os_ref[p0_ref[n]]
                      + g2_ref[2 * n + 1] * os_ref[p1_ref[n]])
        return carry
    jax.lax.fori_loop(0, _N, body, 0)


def kernel(x, gate_w, gate_b, w_gate, w_up, w_down):
    xf = x.reshape(-1, x.shape[-1])
    logits, g2, p0, p1, off, cnt = pl.pallas_call(
        _plan_kernel,
        out_shape=(
            jax.ShapeDtypeStruct((_N, _E), jnp.float32),
            jax.ShapeDtypeStruct((_N, 2), jnp.float32),
            jax.ShapeDtypeStruct((_N, 1), jnp.int32),
            jax.ShapeDtypeStruct((_N, 1), jnp.int32),
            jax.ShapeDtypeStruct((1, _E), jnp.int32),
            jax.ShapeDtypeStruct((1, _E), jnp.int32),
        ),
    )(xf, gate_w, gate_b.reshape(1, _E))

    xb = xf.astype(jnp.bfloat16).reshape(_N, 8, 128)
    xs, wt, we, wv = pl.pallas_call(
        _dispatch_kernel,
        grid_spec=pltpu.PrefetchScalarGridSpec(
            num_scalar_prefetch=4,
            grid=(1,),
            in_specs=[pl.BlockSpec((_N, 8, 128), lambda i, *_: (0, 0, 0))],
            out_specs=[
                pl.BlockSpec((_NK, 8, 128), lambda i, *_: (0, 0, 0)),
                pl.BlockSpec(memory_space=pltpu.SMEM),
                pl.BlockSpec(memory_space=pltpu.SMEM),
                pl.BlockSpec(memory_space=pltpu.SMEM),
            ],
        ),
        out_shape=(
            jax.ShapeDtypeStruct((_NK, 8, 128), jnp.bfloat16),
            jax.ShapeDtypeStruct((_W,), jnp.int32),
            jax.ShapeDtypeStruct((_W,), jnp.int32),
            jax.ShapeDtypeStruct((_W,), jnp.int32),
        ),
    )(p0.reshape(_N), p1.reshape(_N), off.reshape(_E), cnt.reshape(_E), xb)

    wg = w_gate.astype(jnp.bfloat16)
    wu = w_up.astype(jnp.bfloat16)
    wd = w_down.astype(jnp.bfloat16)
    outs = pl.pallas_call(
        _ffn_kernel,
        grid_spec=pltpu.PrefetchScalarGridSpec(
            num_scalar_prefetch=5,
            grid=(_W, _F),
            in_specs=[
                pl.BlockSpec((_TILE, _D), lambda w, f, wt, we, wv, o, c: (wt[w], 0)),
                pl.BlockSpec((1, _FFT, _D), lambda w, f, wt, we, wv, o, c: (we[w], f, 0)),
                pl.BlockSpec((1, _FFT, _D), lambda w, f, wt, we, wv, o, c: (we[w], f, 0)),
                pl.BlockSpec((1, _D, _FFT), lambda w, f, wt, we, wv, o, c: (we[w], 0, f)),
            ],
            out_specs=pl.BlockSpec(
                (_TILE, _D), lambda w, f, wt, we, wv, o, c: (wt[w], 0)),
        ),
        out_shape=jax.ShapeDtypeStruct((_NK, _D), jnp.float32),
    )(wt, we, wv, off.reshape(_E), cnt.reshape(_E), xs.reshape(_NK, _D), wg, wu, wd)

    final = pl.pallas_call(
        _combine_kernel,
        grid_spec=pltpu.PrefetchScalarGridSpec(
            num_scalar_prefetch=3,
            grid=(1,),
            in_specs=[pl.BlockSpec((_NK, 8, 128), lambda i, *_: (0, 0, 0))],
            out_specs=pl.BlockSpec((_N, 8, 128), lambda i, *_: (0, 0, 0)),
        ),
        out_shape=jax.ShapeDtypeStruct((_N, 8, 128), jnp.float32),
    )(p0.reshape(_N), p1.reshape(_N), g2.reshape(_N * 2), outs.reshape(_NK, 8, 128))
    return final.reshape(x.shape), logits


# speedup vs baseline: 1.4502x; 1.1838x over previous
import jax
import jax.numpy as jnp
from jax.experimental import pallas as pl
from jax.experimental.pallas import tpu as pltpu

_E, _D, _FF, _K = 8, 1024, 2048, 2
_N = 2048
_NK = _N * _K
_TILE = 512
_T = _NK // _TILE
_W = _T + _E - 1
_FFT = 512
_F = _FF // _FFT


def _cumsum(v, axis):
    n = v.shape[axis]
    sh = 1
    while sh < n:
        z = jnp.zeros_like(jax.lax.slice_in_dim(v, 0, sh, axis=axis))
        v = v + jnp.concatenate(
            [z, jax.lax.slice_in_dim(v, 0, n - sh, axis=axis)], axis=axis)
        sh *= 2
    return v


def _plan_kernel(x_ref, gw_ref, gb_ref,
                 logits_ref, g2_ref, p01_ref, off_ref, cnt_ref):
    x = x_ref[...]
    logits = jax.lax.dot_general(
        x, gw_ref[...], (((1,), (1,)), ((), ())),
        preferred_element_type=jnp.float32) + gb_ref[...]
    logits_ref[...] = logits
    lt = logits.T
    sub = jax.lax.broadcasted_iota(jnp.int32, lt.shape, 0)
    l0 = jnp.max(lt, axis=0, keepdims=True)
    i0 = jnp.min(jnp.where(lt == l0, sub, _E), axis=0, keepdims=True)
    masked = jnp.where(sub == i0, -jnp.inf, lt)
    l1 = jnp.max(masked, axis=0, keepdims=True)
    i1 = jnp.min(jnp.where(masked == l1, sub, _E), axis=0, keepdims=True)
    g0 = jax.nn.sigmoid(l0 - l1)
    g2_ref[...] = jnp.concatenate([g0, 1.0 - g0], axis=0)
    m0 = (sub == i0).astype(jnp.int32)
    m1 = (sub == i1).astype(jnp.int32)
    c0i = _cumsum(m0, 1)
    c1i = _cumsum(m1, 1)
    cnt0 = c0i[:, _N - 1:_N]
    cnt1 = c1i[:, _N - 1:_N]
    counts = cnt0 + cnt1
    off = _cumsum(counts, 0) - counts
    off_ref[...] = off
    cnt_ref[...] = counts
    p0 = jnp.sum(m0 * (off + c0i - m0), axis=0, keepdims=True)
    p1 = jnp.sum(m1 * (off + cnt0 + c1i - m1), axis=0, keepdims=True)
    p01_ref[...] = jnp.concatenate([p0, p1], axis=0)


def _dispatch_kernel(p01_ref, off_ref, cnt_ref,
                     x_ref, xs_ref, wt_ref, we_ref, wv_ref):
    def body(j, carry):
        row = x_ref[j]
        xs_ref[p01_ref[j]] = row
        xs_ref[p01_ref[_N + j]] = row
        return carry
    jax.lax.fori_loop(0, _N, body, 0)

    def wbody(p, idx):
        t = p // _E
        e = p % _E
        s = off_ref[e]
        c = cnt_ref[e]
        active = jnp.logical_and(
            jnp.logical_and(s < (t + 1) * _TILE, s + c > t * _TILE), c > 0)

        @pl.when(active)
        def _():
            wt_ref[idx] = t
            we_ref[idx] = e
            wv_ref[idx] = 1

        return idx + active.astype(jnp.int32)

    nitems = jax.lax.fori_loop(0, _T * _E, wbody, 0)

    def pbody(i, carry):
        @pl.when(i >= nitems)
        def _():
            wt_ref[i] = wt_ref[nitems - 1]
            we_ref[i] = we_ref[nitems - 1]
            wv_ref[i] = 0
        return carry

    jax.lax.fori_loop(0, _W, pbody, 0)


def _ffn_kernel(wt_ref, we_ref, wv_ref, off_ref, cnt_ref,
                xs_ref, wg_ref, wu_ref, wd_ref, out_ref):
    w = pl.program_id(0)
    f = pl.program_id(1)
    t = wt_ref[w]
    e = we_ref[w]
    first = jnp.logical_or(w == 0, t != wt_ref[jnp.maximum(w - 1, 0)])

    @pl.when(jnp.logical_and(first, f == 0))
    def _():
        out_ref[...] = jnp.zeros_like(out_ref)

    @pl.when(wv_ref[w] == 1)
    def _():
        x = xs_ref[...]
        a = jax.lax.dot_general(x, wg_ref[0], (((1,), (1,)), ((), ())),
                                preferred_element_type=jnp.float32)
        b = jax.lax.dot_general(x, wu_ref[0], (((1,), (1,)), ((), ())),
                                preferred_element_type=jnp.float32)
        h = (a * jax.nn.sigmoid(a)) * b
        row = t * _TILE + jax.lax.broadcasted_iota(jnp.int32, (_TILE, 1), 0)
        s = off_ref[e]
        mask = jnp.logical_and(row >= s, row < s + cnt_ref[e])
        hm = jnp.where(mask, h, 0.0)
        out_ref[...] += jax.lax.dot_general(
            hm, wd_ref[0], (((1,), (1,)), ((), ())),
            preferred_element_type=jnp.float32)


def _combine_kernel(p01_ref, g2_ref, os_ref, out_ref):
    def body(n, carry):
        out_ref[n] = (g2_ref[n] * os_ref[p01_ref[n]]
                      + g2_ref[_N + n] * os_ref[p01_ref[_N + n]])
        return carry
    jax.lax.fori_loop(0, _N, body, 0)


def kernel(x, gate_w, gate_b, w_gate, w_up, w_down):
    xf = x.reshape(-1, x.shape[-1])
    logits, g2, p01, off, cnt = pl.pallas_call(
        _plan_kernel,
        out_shape=(
            jax.ShapeDtypeStruct((_N, _E), jnp.float32),
            jax.ShapeDtypeStruct((2, _N), jnp.float32),
            jax.ShapeDtypeStruct((2, _N), jnp.int32),
            jax.ShapeDtypeStruct((_E, 1), jnp.int32),
            jax.ShapeDtypeStruct((_E, 1), jnp.int32),
        ),
    )(xf, gate_w, gate_b.reshape(1, _E))

    x3 = xf.reshape(_N, 8, 128)
    xs, wt, we, wv = pl.pallas_call(
        _dispatch_kernel,
        grid_spec=pltpu.PrefetchScalarGridSpec(
            num_scalar_prefetch=3,
            grid=(1,),
            in_specs=[pl.BlockSpec((_N, 8, 128), lambda i, *_: (0, 0, 0))],
            out_specs=[
                pl.BlockSpec((_NK, 8, 128), lambda i, *_: (0, 0, 0)),
                pl.BlockSpec(memory_space=pltpu.SMEM),
                pl.BlockSpec(memory_space=pltpu.SMEM),
                pl.BlockSpec(memory_space=pltpu.SMEM),
            ],
        ),
        out_shape=(
            jax.ShapeDtypeStruct((_NK, 8, 128), jnp.float32),
            jax.ShapeDtypeStruct((_W,), jnp.int32),
            jax.ShapeDtypeStruct((_W,), jnp.int32),
            jax.ShapeDtypeStruct((_W,), jnp.int32),
        ),
    )(p01.reshape(_NK), off.reshape(_E), cnt.reshape(_E), x3)

    outs = pl.pallas_call(
        _ffn_kernel,
        grid_spec=pltpu.PrefetchScalarGridSpec(
            num_scalar_prefetch=5,
            grid=(_W, _F),
            in_specs=[
                pl.BlockSpec((_TILE, _D), lambda w, f, wt, we, wv, o, c: (wt[w], 0)),
                pl.BlockSpec((1, _FFT, _D), lambda w, f, wt, we, wv, o, c: (we[w], f, 0)),
                pl.BlockSpec((1, _FFT, _D), lambda w, f, wt, we, wv, o, c: (we[w], f, 0)),
                pl.BlockSpec((1, _D, _FFT), lambda w, f, wt, we, wv, o, c: (we[w], 0, f)),
            ],
            out_specs=pl.BlockSpec(
                (_TILE, _D), lambda w, f, wt, we, wv, o, c: (wt[w], 0)),
        ),
        out_shape=jax.ShapeDtypeStruct((_NK, _D), jnp.float32),
    )(wt, we, wv, off.reshape(_E), cnt.reshape(_E),
      xs.reshape(_NK, _D), w_gate, w_up, w_down)

    final = pl.pallas_call(
        _combine_kernel,
        grid_spec=pltpu.PrefetchScalarGridSpec(
            num_scalar_prefetch=2,
            grid=(1,),
            in_specs=[pl.BlockSpec((_NK, 8, 128), lambda i, *_: (0, 0, 0))],
            out_specs=pl.BlockSpec((_N, 8, 128), lambda i, *_: (0, 0, 0)),
        ),
        out_shape=jax.ShapeDtypeStruct((_N, 8, 128), jnp.float32),
    )(p01.reshape(_NK), g2.reshape(_NK), outs.reshape(_NK, 8, 128))
    return final.reshape(x.shape), logits


# speedup vs baseline: 1.6852x; 1.1621x over previous
import jax
import jax.numpy as jnp
from jax.experimental import pallas as pl
from jax.experimental.pallas import tpu as pltpu

_E, _D, _FF, _K = 8, 1024, 2048, 2
_N = 2048
_NK = _N * _K
_TILE = 512
_T = _NK // _TILE
_W = _T + _E - 1
_FFT = 512
_F = _FF // _FFT


def _cumsum(v, axis):
    n = v.shape[axis]
    sh = 1
    while sh < n:
        z = jnp.zeros_like(jax.lax.slice_in_dim(v, 0, sh, axis=axis))
        v = v + jnp.concatenate(
            [z, jax.lax.slice_in_dim(v, 0, n - sh, axis=axis)], axis=axis)
        sh *= 2
    return v


def _plan_kernel(x_ref, gw_ref, gb_ref,
                 logits_ref, x3_ref, g2_ref, p01_ref, off_ref, cnt_ref):
    x = x_ref[...]
    x3_ref[...] = x.reshape(_N, 8, 128)
    logits = jax.lax.dot_general(
        x, gw_ref[...], (((1,), (1,)), ((), ())),
        preferred_element_type=jnp.float32) + gb_ref[...]
    logits_ref[...] = logits
    lt = logits.T
    sub = jax.lax.broadcasted_iota(jnp.int32, lt.shape, 0)
    l0 = jnp.max(lt, axis=0, keepdims=True)
    i0 = jnp.min(jnp.where(lt == l0, sub, _E), axis=0, keepdims=True)
    masked = jnp.where(sub == i0, -jnp.inf, lt)
    l1 = jnp.max(masked, axis=0, keepdims=True)
    i1 = jnp.min(jnp.where(masked == l1, sub, _E), axis=0, keepdims=True)
    g0 = jax.nn.sigmoid(l0 - l1)
    g2_ref[...] = jnp.concatenate([g0, 1.0 - g0], axis=0)
    m0 = (sub == i0).astype(jnp.int32)
    m1 = (sub == i1).astype(jnp.int32)
    c0i = _cumsum(m0, 1)
    c1i = _cumsum(m1, 1)
    cnt0 = c0i[:, _N - 1:_N]
    cnt1 = c1i[:, _N - 1:_N]
    counts = cnt0 + cnt1
    off = _cumsum(counts, 0) - counts
    off_ref[...] = off
    cnt_ref[...] = counts
    p0 = jnp.sum(m0 * (off + c0i - m0), axis=0, keepdims=True)
    p1 = jnp.sum(m1 * (off + cnt0 + c1i - m1), axis=0, keepdims=True)
    p01_ref[...] = jnp.concatenate([p0, p1], axis=0)


def _dispatch_kernel(p01_ref, off_ref, cnt_ref,
                     x_ref, xs_ref, wt_ref, we_ref, wv_ref):
    def body(j, carry):
        row = x_ref[j]
        xs_ref[p01_ref[0, j]] = row
        xs_ref[p01_ref[1, j]] = row
        return carry
    jax.lax.fori_loop(0, _N, body, 0)

    def wbody(p, idx):
        t = p // _E
        e = p % _E
        s = off_ref[e, 0]
        c = cnt_ref[e, 0]
        active = jnp.logical_and(
            jnp.logical_and(s < (t + 1) * _TILE, s + c > t * _TILE), c > 0)

        @pl.when(active)
        def _():
            wt_ref[idx] = t
            we_ref[idx] = e
            wv_ref[idx] = 1

        return idx + active.astype(jnp.int32)

    nitems = jax.lax.fori_loop(0, _T * _E, wbody, 0)

    def pbody(i, carry):
        @pl.when(i >= nitems)
        def _():
            wt_ref[i] = wt_ref[nitems - 1]
            we_ref[i] = we_ref[nitems - 1]
            wv_ref[i] = 0
        return carry

    jax.lax.fori_loop(0, _W, pbody, 0)


def _ffn_kernel(wt_ref, we_ref, wv_ref, off_ref, cnt_ref,
                xs_ref, wg_ref, wu_ref, wd_ref, out_ref,
                x2d_ref, acc_ref):
    w = pl.program_id(0)
    f = pl.program_id(1)
    t = wt_ref[w]
    e = we_ref[w]
    first = jnp.logical_or(w == 0, t != wt_ref[jnp.maximum(w - 1, 0)])
    last = jnp.logical_and(
        f == _F - 1,
        jnp.logical_or(w == _W - 1, wt_ref[jnp.minimum(w + 1, _W - 1)] != t))

    @pl.when(jnp.logical_and(first, f == 0))
    def _():
        x2d_ref[...] = xs_ref[...].reshape(_TILE, _D)
        acc_ref[...] = jnp.zeros_like(acc_ref)

    @pl.when(wv_ref[w] == 1)
    def _():
        x = x2d_ref[...]
        a = jax.lax.dot_general(x, wg_ref[0], (((1,), (1,)), ((), ())),
                                preferred_element_type=jnp.float32)
        b = jax.lax.dot_general(x, wu_ref[0], (((1,), (1,)), ((), ())),
                                preferred_element_type=jnp.float32)
        h = (a * jax.nn.sigmoid(a)) * b
        row = t * _TILE + jax.lax.broadcasted_iota(jnp.int32, (_TILE, 1), 0)
        s = off_ref[e, 0]
        mask = jnp.logical_and(row >= s, row < s + cnt_ref[e, 0])
        hm = jnp.where(mask, h, 0.0)
        acc_ref[...] += jax.lax.dot_general(
            hm, wd_ref[0], (((1,), (1,)), ((), ())),
            preferred_element_type=jnp.float32)

    @pl.when(last)
    def _():
        out_ref[...] = acc_ref[...].reshape(_TILE, 8, 128)


def _combine_kernel(p01_ref, g2_ref, os_ref, out_ref):
    def body(n, carry):
        out_ref[n] = (g2_ref[0, n] * os_ref[p01_ref[0, n]]
                      + g2_ref[1, n] * os_ref[p01_ref[1, n]])
        return carry
    jax.lax.fori_loop(0, _N, body, 0)


def kernel(x, gate_w, gate_b, w_gate, w_up, w_down):
    xf = x.reshape(-1, x.shape[-1])
    logits, x3, g2, p01, off, cnt = pl.pallas_call(
        _plan_kernel,
        out_shape=(
            jax.ShapeDtypeStruct((_N, _E), jnp.float32),
            jax.ShapeDtypeStruct((_N, 8, 128), jnp.float32),
            jax.ShapeDtypeStruct((2, _N), jnp.float32),
            jax.ShapeDtypeStruct((2, _N), jnp.int32),
            jax.ShapeDtypeStruct((_E, 1), jnp.int32),
            jax.ShapeDtypeStruct((_E, 1), jnp.int32),
        ),
    )(xf, gate_w, gate_b.reshape(1, _E))

    xs, wt, we, wv = pl.pallas_call(
        _dispatch_kernel,
        grid_spec=pltpu.PrefetchScalarGridSpec(
            num_scalar_prefetch=3,
            grid=(1,),
            in_specs=[pl.BlockSpec((_N, 8, 128), lambda i, *_: (0, 0, 0))],
            out_specs=[
                pl.BlockSpec((_NK, 8, 128), lambda i, *_: (0, 0, 0)),
                pl.BlockSpec(memory_space=pltpu.SMEM),
                pl.BlockSpec(memory_space=pltpu.SMEM),
                pl.BlockSpec(memory_space=pltpu.SMEM),
            ],
        ),
        out_shape=(
            jax.ShapeDtypeStruct((_NK, 8, 128), jnp.float32),
            jax.ShapeDtypeStruct((_W,), jnp.int32),
            jax.ShapeDtypeStruct((_W,), jnp.int32),
            jax.ShapeDtypeStruct((_W,), jnp.int32),
        ),
    )(p01, off, cnt, x3)

    outs = pl.pallas_call(
        _ffn_kernel,
        grid_spec=pltpu.PrefetchScalarGridSpec(
            num_scalar_prefetch=5,
            grid=(_W, _F),
            in_specs=[
                pl.BlockSpec((_TILE, 8, 128),
                             lambda w, f, wt, we, wv, o, c: (wt[w], 0, 0)),
                pl.BlockSpec((1, _FFT, _D),
                             lambda w, f, wt, we, wv, o, c: (we[w], f, 0)),
                pl.BlockSpec((1, _FFT, _D),
                             lambda w, f, wt, we, wv, o, c: (we[w], f, 0)),
                pl.BlockSpec((1, _D, _FFT),
                             lambda w, f, wt, we, wv, o, c: (we[w], 0, f)),
            ],
            out_specs=pl.BlockSpec(
                (_TILE, 8, 128), lambda w, f, wt, we, wv, o, c: (wt[w], 0, 0)),
            scratch_shapes=[
                pltpu.VMEM((_TILE, _D), jnp.float32),
                pltpu.VMEM((_TILE, _D), jnp.float32),
            ],
        ),
        out_shape=jax.ShapeDtypeStruct((_NK, 8, 128), jnp.float32),
    )(wt, we, wv, off, cnt, xs, w_gate, w_up, w_down)

    final = pl.pallas_call(
        _combine_kernel,
        grid_spec=pltpu.PrefetchScalarGridSpec(
            num_scalar_prefetch=2,
            grid=(1,),
            in_specs=[pl.BlockSpec((_NK, 8, 128), lambda i, *_: (0, 0, 0))],
            out_specs=pl.BlockSpec((_N, 8, 128), lambda i, *_: (0, 0, 0)),
        ),
        out_shape=jax.ShapeDtypeStruct((_N, 8, 128), jnp.float32),
    )(p01, g2, outs)
    return final.reshape(x.shape), logits
